# Initial kernel scaffold; baseline (speedup 1.0000x reference)
import jax
import jax.numpy as jnp
from jax.experimental import pallas as pl


def kernel(id, ttaEncoding):
    raise NotImplementedError("write your pallas kernel here")


# speedup vs baseline: 3.2505x; 3.2505x over previous
import functools

import jax
import jax.numpy as jnp
from jax import lax
from jax.experimental import pallas as pl
from jax.experimental.pallas import tpu as pltpu
from jax.experimental.pallas import tpu_sc as plsc

B0, B1 = 16384, 20
D = 128
B = B0 * B1
NC, NS = 2, 16
NW = NC * NS
BPW = B // NW
IW = 128
C = 256
NI = C // IW
NCHUNK = BPW // C
IDX_ROWS_PER_CHUNK = C // IW

_mesh = plsc.VectorSubcoreMesh(core_axis_name="c", subcore_axis_name="s")


@functools.partial(
    pl.kernel,
    mesh=_mesh,
    out_type=jax.ShapeDtypeStruct((B, D), jnp.float32),
    scratch_types=[
        pltpu.VMEM((BPW // IW, IW), jnp.int32),
        pltpu.VMEM((C, D), jnp.float32),
        pltpu.SemaphoreType.DMA,
    ],
)
def _gather(table_hbm, idx_hbm, out_hbm, idx_v, rows_v, sem):
    wid = lax.axis_index("s") * NC + lax.axis_index("c")
    row_base = pl.multiple_of(wid * BPW, BPW)
    irow_base = pl.multiple_of(row_base // IW, BPW // IW)

    pltpu.sync_copy(idx_hbm.at[pl.ds(irow_base, BPW // IW)], idx_v)

    def body(g, carry):
        off = pl.multiple_of(row_base + g * C, C)
        descs = [
            pltpu.async_copy(
                table_hbm.at[idx_v.at[g * NI + j]],
                rows_v.at[pl.ds(j * IW, IW)],
                sem,
            )
            for j in range(NI)
        ]
        for d in descs:
            d.wait()
        pltpu.sync_copy(rows_v, out_hbm.at[pl.ds(off, C)])
        return carry

    lax.fori_loop(0, NCHUNK, body, 0)


def kernel(id, ttaEncoding):
    idx2d = id.astype(jnp.int32).reshape(B // IW, IW)
    out = _gather(ttaEncoding, idx2d)
    return out.reshape(B0, B1, D)


# speedup vs baseline: 3.4615x; 1.0649x over previous
import functools

import jax
import jax.numpy as jnp
from jax import lax
from jax.experimental import pallas as pl
from jax.experimental.pallas import tpu as pltpu
from jax.experimental.pallas import tpu_sc as plsc

B0, B1 = 16384, 20
D = 128
B = B0 * B1
NC, NS = 2, 16
NW = NC * NS
BPW = B // NW
CW = 128
NCH = BPW // CW
NBUF = 4
NG = NCH // NBUF

_mesh = plsc.VectorSubcoreMesh(core_axis_name="c", subcore_axis_name="s")


@functools.partial(
    pl.kernel,
    mesh=_mesh,
    out_type=jax.ShapeDtypeStruct((B, D), jnp.float32),
    scratch_types=[pltpu.VMEM((NCH, CW), jnp.int32)]
    + [pltpu.VMEM((CW, D), jnp.float32) for _ in range(NBUF)]
    + [pltpu.SemaphoreType.DMA for _ in range(2 * NBUF)],
)
def _gather(table_hbm, idx_hbm, out_hbm, idx_v, *bufs_sems):
    bufs = bufs_sems[:NBUF]
    gsem = bufs_sems[NBUF:2 * NBUF]
    osem = bufs_sems[2 * NBUF:]

    wid = lax.axis_index("s") * NC + lax.axis_index("c")
    row_base = pl.multiple_of(wid * BPW, BPW)
    irow_base = pl.multiple_of(row_base // CW, NCH)

    pltpu.sync_copy(idx_hbm.at[pl.ds(irow_base, NCH)], idx_v)

    def fire_gather(k, b):
        pltpu.async_copy(table_hbm.at[idx_v.at[k]], bufs[b], gsem[b])

    def wait_gather(k, b):
        pltpu.make_async_copy(table_hbm.at[idx_v.at[k]], bufs[b], gsem[b]).wait()

    def fire_wb(k, b):
        off = pl.multiple_of(row_base + k * CW, CW)
        pltpu.async_copy(bufs[b], out_hbm.at[pl.ds(off, CW)], osem[b])

    def wait_wb(k, b):
        off = pl.multiple_of(row_base + k * CW, CW)
        pltpu.make_async_copy(bufs[b], out_hbm.at[pl.ds(off, CW)], osem[b]).wait()

    for b in range(NBUF):
        fire_gather(b, b)

    def ring_round(p, carry):
        for b in range(NBUF):
            k = p * NBUF + b
            wait_gather(k, b)
            fire_wb(k, b)
            wait_wb(k, b)
            fire_gather(k + NBUF, b)
        return carry

    lax.fori_loop(0, NG - 1, ring_round, 0)

    for b in range(NBUF):
        k = (NG - 1) * NBUF + b
        wait_gather(k, b)
        fire_wb(k, b)
    for b in range(NBUF):
        k = (NG - 1) * NBUF + b
        wait_wb(k, b)


def kernel(id, ttaEncoding):
    idx2d = id.astype(jnp.int32).reshape(B // CW, CW)
    out = _gather(ttaEncoding, idx2d)
    return out.reshape(B0, B1, D)


# speedup vs baseline: 6.0383x; 1.7444x over previous
import functools

import jax
import jax.numpy as jnp
from jax import lax
from jax.experimental import pallas as pl
from jax.experimental.pallas import tpu as pltpu
from jax.experimental.pallas import tpu_sc as plsc

B0, B1 = 16384, 20
D = 128
B = B0 * B1
NC, NS = 2, 16
NW = NC * NS
BPW = B // NW
GPW = B0 // NW
G = 4
CW = G * B1
NCH = GPW // G
NBUF = 4
NG = NCH // NBUF

_mesh = plsc.VectorSubcoreMesh(core_axis_name="c", subcore_axis_name="s")


@functools.partial(
    pl.kernel,
    mesh=_mesh,
    out_type=jax.ShapeDtypeStruct((B0, B1, D), jnp.float32),
    scratch_types=[pltpu.VMEM((GPW, B1), jnp.int32)]
    + [pltpu.VMEM((G, B1, D), jnp.float32) for _ in range(NBUF)]
    + [pltpu.SemaphoreType.DMA for _ in range(2 * NBUF)],
)
def _gather(table_hbm, idx_hbm, out_hbm, idx_v, *bufs_sems):
    bufs = bufs_sems[:NBUF]
    gsem = bufs_sems[NBUF:2 * NBUF]
    osem = bufs_sems[2 * NBUF:]

    wid = lax.axis_index("s") * NC + lax.axis_index("c")
    grp_base = pl.multiple_of(wid * GPW, GPW)

    pltpu.sync_copy(idx_hbm.at[pl.ds(grp_base, GPW)], idx_v)

    def fire_gather(k, b):
        for g in range(G):
            pltpu.async_copy(
                table_hbm.at[idx_v.at[k * G + g]],
                bufs[b].at[g],
                gsem[b],
            )

    def wait_gather(k, b):
        for g in range(G):
            pltpu.make_async_copy(
                table_hbm.at[idx_v.at[k * G + g]],
                bufs[b].at[g],
                gsem[b],
            ).wait()

    def fire_wb(k, b):
        pltpu.async_copy(bufs[b], out_hbm.at[pl.ds(grp_base + k * G, G)], osem[b])

    def wait_wb(k, b):
        pltpu.make_async_copy(
            bufs[b], out_hbm.at[pl.ds(grp_base + k * G, G)], osem[b]
        ).wait()

    for b in range(NBUF):
        fire_gather(b, b)

    def ring_round(p, carry):
        for b in range(NBUF):
            k = p * NBUF + b
            wait_gather(k, b)
            fire_wb(k, b)
            wait_wb(k, b)
            fire_gather(k + NBUF, b)
        return carry

    lax.fori_loop(0, NG - 1, ring_round, 0)

    for b in range(NBUF):
        k = (NG - 1) * NBUF + b
        wait_gather(k, b)
        fire_wb(k, b)
    for b in range(NBUF):
        k = (NG - 1) * NBUF + b
        wait_wb(k, b)


def kernel(id, ttaEncoding):
    return _gather(ttaEncoding, id.astype(jnp.int32))
